# baseline (speedup 1.0000x reference)
import jax
import jax.numpy as jnp
from jax.experimental import pallas as pl
from jax.experimental.pallas import tpu as pltpu

NUM_EXPERTS_ = 8


def _moe_body(x_ref, disp_ref, comb_ref, w_ref, b_ref, out_ref):
    xd = jax.lax.dot_general(
        disp_ref[...], x_ref[...],
        (((0,), (0,)), ((), ())),
        preferred_element_type=jnp.float32,
    )
    y = jax.lax.dot_general(
        xd, w_ref[0],
        (((1,), (1,)), ((), ())),
        preferred_element_type=jnp.float32,
    )
    y = y + b_ref[...]
    contrib = jnp.dot(comb_ref[...], y, preferred_element_type=jnp.float32)

    @pl.when(pl.program_id(0) == 0)
    def _init():
        out_ref[...] = contrib

    @pl.when(pl.program_id(0) != 0)
    def _acc():
        out_ref[...] += contrib


def kernel(x, combine_array, dispatch_mask, W, b):
    B, T, E, I = x.shape
    C = combine_array.shape[-1]
    O = W.shape[0]
    x2 = x.reshape(T, E * I)
    comb2 = combine_array.reshape(T, E * C)
    disp2 = dispatch_mask.reshape(T, E * C)
    w3 = W.reshape(E, O, I)
    b2 = b.reshape(1, O)

    out = pl.pallas_call(
        _moe_body,
        grid=(E,),
        in_specs=[
            pl.BlockSpec((T, I), lambda e: (0, e)),
            pl.BlockSpec((T, C), lambda e: (0, e)),
            pl.BlockSpec((T, C), lambda e: (0, e)),
            pl.BlockSpec((1, O, I), lambda e: (e, 0, 0)),
            pl.BlockSpec((1, O), lambda e: (0, 0)),
        ],
        out_specs=pl.BlockSpec((T, O), lambda e: (0, 0)),
        out_shape=jax.ShapeDtypeStruct((T, O), jnp.float32),
        compiler_params=pltpu.CompilerParams(
            dimension_semantics=("arbitrary",),
        ),
    )(x2, disp2, comb2, w3, b2)
    return out.reshape(B, T, O)
